# Initial kernel scaffold; baseline (speedup 1.0000x reference)
import jax
import jax.numpy as jnp
from jax.experimental import pallas as pl


def kernel(z, edge_index, W1, att_src1, att_dst1, b1, W2, att_src2, att_dst2, b2, Wlin, blin):
    raise NotImplementedError("write your pallas kernel here")


# speedup vs baseline: 61.6204x; 61.6204x over previous
import functools

import jax
import jax.numpy as jnp
from jax import lax
from jax.experimental import pallas as pl
from jax.experimental.pallas import tpu as pltpu
from jax.experimental.pallas import tpu_sc as plsc

N = 10000
E = 320000
EMB = 128
HID = 16
HEADS = 8
F = HEADS * HID

NW = 32
PER_W = E // NW
C = 80
NCH = PER_W // C
ZTILES = 10
RPT = N // ZTILES
ZROWS = 40
DEN_R = 632

BR = 1000
GRID = N // BR


def _proj_body(x_ref, w_ref, acat_ref, h_ref, ac_ref):
    h = jnp.dot(x_ref[...], w_ref[...], preferred_element_type=jnp.float32)
    h_ref[...] = h
    ac_ref[...] = jnp.dot(h, acat_ref[...], preferred_element_type=jnp.float32)


def _project(x, W, Acat):
    return pl.pallas_call(
        _proj_body,
        grid=(GRID,),
        in_specs=[
            pl.BlockSpec((BR, EMB), lambda i: (i, 0)),
            pl.BlockSpec((EMB, F), lambda i: (0, 0)),
            pl.BlockSpec((F, F), lambda i: (0, 0)),
        ],
        out_specs=[
            pl.BlockSpec((BR, F), lambda i: (i, 0)),
            pl.BlockSpec((BR, F), lambda i: (i, 0)),
        ],
        out_shape=[
            jax.ShapeDtypeStruct((N, F), jnp.float32),
            jax.ShapeDtypeStruct((N, F), jnp.float32),
        ],
    )(x, W, Acat)


def _combine_body(np0, np1, dp0, dp1, h, ac, r16, bvec, wn, acatn, bn,
                  hn_ref, acn_ref):
    ac_ = ac[...]
    asum = ac_[:, 0:8] + ac_[:, 8:16]
    es = jnp.exp(jnp.maximum(asum, 0.2 * asum))
    den8 = dp0[...] + dp1[...] + es + 1e-16
    es_b = jnp.dot(es, r16[...], preferred_element_type=jnp.float32)
    numer = np0[...] + np1[...] + es_b * h[...]
    den_b = jnp.dot(den8, r16[...], preferred_element_type=jnp.float32)
    x = jnp.tanh(numer / den_b + bvec[...])
    hn = jnp.dot(x, wn[...], preferred_element_type=jnp.float32) + bn[...]
    hn_ref[...] = hn
    acn_ref[...] = jnp.dot(hn, acatn[...], preferred_element_type=jnp.float32)


def _combine_project(np0, np1, dp0, dp1, h, ac, R16, b, Wn, Acatn, bn):
    K = Wn.shape[1]
    return pl.pallas_call(
        _combine_body,
        grid=(GRID,),
        in_specs=[
            pl.BlockSpec((BR, F), lambda i: (i, 0)),
            pl.BlockSpec((BR, F), lambda i: (i, 0)),
            pl.BlockSpec((BR, 8), lambda i: (i, 0)),
            pl.BlockSpec((BR, 8), lambda i: (i, 0)),
            pl.BlockSpec((BR, F), lambda i: (i, 0)),
            pl.BlockSpec((BR, F), lambda i: (i, 0)),
            pl.BlockSpec((HEADS, F), lambda i: (0, 0)),
            pl.BlockSpec((1, F), lambda i: (0, 0)),
            pl.BlockSpec((F, K), lambda i: (0, 0)),
            pl.BlockSpec((K, F), lambda i: (0, 0)),
            pl.BlockSpec((1, K), lambda i: (0, 0)),
        ],
        out_specs=[
            pl.BlockSpec((BR, K), lambda i: (i, 0)),
            pl.BlockSpec((BR, F), lambda i: (i, 0)),
        ],
        out_shape=[
            jax.ShapeDtypeStruct((N, K), jnp.float32),
            jax.ShapeDtypeStruct((N, F), jnp.float32),
        ],
    )(np0, np1, dp0, dp1, h, ac, R16, b, Wn, Acatn, bn)


_GATHER_DNUMS = lax.GatherDimensionNumbers(
    offset_dims=(), collapsed_slice_dims=(0,), start_index_map=(0,))


def _dyn_gather(x, idx):
    return lax.gather(x, idx[:, None], _GATHER_DNUMS, slice_sizes=(1,),
                      mode=lax.GatherScatterMode.PROMISE_IN_BOUNDS)


def _edge_body(h_hbm, ac_hbm, src_hbm, dst_hbm, num_out, den_out,
               src_v, dst_v, dstrow_v, asrc_v, adst_v, h_v, zbuf,
               num_sh, den_sh, sem_a, sem_b, sem_h):
    c = lax.axis_index("c")
    s = lax.axis_index("s")
    wid = c * 16 + s
    lane = lax.iota(jnp.int32, 16)
    zeros16 = jnp.zeros((16,), jnp.float32)

    @pl.when(s < ZTILES)
    def _zero_src():
        def zrow(r, _):
            for j in range(F // 16):
                zbuf[r, pl.ds(16 * j, 16)] = zeros16
            return 0
        lax.fori_loop(0, ZROWS, zrow, 0)
    row0 = s * RPT

    @pl.when(s < ZTILES)
    def _zero_num():
        for p in range(RPT // ZROWS):
            pltpu.sync_copy(zbuf, num_sh.at[pl.ds(row0 + ZROWS * p, ZROWS)])

    @pl.when(s == ZTILES)
    def _zero_den():
        def zrow(r, _):
            for j in range(F // 16):
                zbuf[r, pl.ds(16 * j, 16)] = zeros16
            return 0
        lax.fori_loop(0, ZROWS, zrow, 0)
        for p in range(DEN_R // ZROWS):
            pltpu.sync_copy(zbuf, den_sh.at[pl.ds(ZROWS * p, ZROWS)])
        rem = DEN_R - (DEN_R // ZROWS) * ZROWS
        if rem:
            pltpu.sync_copy(zbuf.at[pl.ds(0, rem)],
                            den_sh.at[pl.ds(DEN_R - rem, rem)])
    plsc.subcore_barrier()

    col8 = (lane & 7) + 8

    def chunk_body(k, _):
        base = wid * PER_W + k * C
        pltpu.sync_copy(src_hbm.at[pl.ds(base, C)], src_v)
        pltpu.sync_copy(dst_hbm.at[pl.ds(base, C)], dst_v)
        cp_a = pltpu.async_copy(ac_hbm.at[src_v], asrc_v, sem_a)
        cp_b = pltpu.async_copy(ac_hbm.at[dst_v], adst_v, sem_b)
        cp_h = pltpu.async_copy(h_hbm.at[src_v], h_v, sem_h)

        def drow(i, _):
            v = dst_v[pl.ds(16 * i, 16)]
            dstrow_v[pl.ds(16 * i, 16)] = lax.shift_right_logical(v, 4)
            return 0
        lax.fori_loop(0, C // 16, drow, 0)
        cp_a.wait()
        cp_b.wait()
        cp_h.wait()

        def group_body(g, _):
            dst16 = dst_v[pl.ds(16 * g, 16)]
            off16v = (dst16 & 14) * 8
            shiftv = (dst16 & 1) * 8
            for j in range(16):
                r = 16 * g + j
                va = asrc_v[r, pl.ds(0, 16)]
                vb = adst_v[r, pl.ds(0, 16)]
                vb8 = _dyn_gather(vb, col8)
                e = va + vb8
                e = jnp.maximum(e, 0.2 * e)
                exv = jnp.exp(e)
                ex16 = jnp.where(lane < 8, exv, 0.0)
                asrc_v[r, pl.ds(0, 16)] = zeros16
                asrc_v[r, pl.ds(off16v[j], 16)] = _dyn_gather(
                    ex16, (lane - shiftv[j]) & 15)
                for head in range(HEADS):
                    bc = _dyn_gather(exv, jnp.full((16,), head, jnp.int32))
                    hv = h_v[r, pl.ds(16 * head, 16)]
                    h_v[r, pl.ds(16 * head, 16)] = hv * bc
            return 0
        lax.fori_loop(0, C // 16, group_body, 0)

        pltpu.sync_copy(asrc_v, den_sh.at[dstrow_v], add=True)
        pltpu.sync_copy(h_v, num_sh.at[dst_v], add=True)

        return 0
    lax.fori_loop(0, NCH, chunk_body, 0)

    plsc.subcore_barrier()

    @pl.when(s < ZTILES)
    def _copy_num():
        pltpu.sync_copy(num_sh.at[pl.ds(row0, RPT)], num_out.at[c, pl.ds(row0, RPT)])

    @pl.when(s == ZTILES)
    def _copy_den():
        pltpu.sync_copy(den_sh, den_out.at[c])


_edge_pass = functools.partial(
    pl.kernel,
    out_type=[
        jax.ShapeDtypeStruct((2, N, F), jnp.float32),
        jax.ShapeDtypeStruct((2, DEN_R, F), jnp.float32),
    ],
    mesh=plsc.VectorSubcoreMesh(core_axis_name="c", subcore_axis_name="s"),
    scratch_types=[
        pltpu.VMEM((C,), jnp.int32),
        pltpu.VMEM((C,), jnp.int32),
        pltpu.VMEM((C,), jnp.int32),
        pltpu.VMEM((C, F), jnp.float32),
        pltpu.VMEM((C, F), jnp.float32),
        pltpu.VMEM((C, F), jnp.float32),
        pltpu.VMEM((ZROWS, F), jnp.float32),
        pltpu.VMEM_SHARED((N, F), jnp.float32),
        pltpu.VMEM_SHARED((DEN_R, F), jnp.float32),
        pltpu.SemaphoreType.DMA,
        pltpu.SemaphoreType.DMA,
        pltpu.SemaphoreType.DMA,
    ],
)(_edge_body)


def _selector(att):
    blk = jnp.kron(jnp.eye(HEADS, dtype=jnp.float32),
                   jnp.ones((HID, 1), jnp.float32))
    return att.reshape(F, 1) * blk


def kernel(z, edge_index, W1, att_src1, att_dst1, b1,
           W2, att_src2, att_dst2, b2, Wlin, blin):
    src = edge_index[0]
    dst = edge_index[1]
    pad = jnp.zeros((F, F - 2 * HEADS), jnp.float32)
    Acat1 = jnp.concatenate([_selector(att_src1), _selector(att_dst1), pad], axis=1)
    Acat2 = jnp.concatenate([_selector(att_src2), _selector(att_dst2), pad], axis=1)
    R16 = jnp.kron(jnp.eye(HEADS, dtype=jnp.float32), jnp.ones((1, HID), jnp.float32))
    zeros_b = jnp.zeros((1, F), jnp.float32)
    acat_dummy = jnp.zeros((1, F), jnp.float32)

    h1, ac1 = _project(z, W1, Acat1)
    n1, d1 = _edge_pass(h1, ac1, src, dst)
    d1 = d1.reshape(2, DEN_R * 16, 8)[:, :N]
    h2, ac2 = _combine_project(n1[0], n1[1], d1[0], d1[1], h1, ac1,
                               R16, b1.reshape(1, F), W2, Acat2, zeros_b)
    n2, d2 = _edge_pass(h2, ac2, src, dst)
    d2 = d2.reshape(2, DEN_R * 16, 8)[:, :N]
    y, _ = _combine_project(n2[0], n2[1], d2[0], d2[1], h2, ac2,
                            R16, b2.reshape(1, F), Wlin, acat_dummy,
                            blin.reshape(1, 1))
    return y
